# baseline (speedup 1.0000x reference)
import functools

import jax
import jax.numpy as jnp
from jax import lax
from jax.experimental import pallas as pl
from jax.experimental.pallas import tpu as pltpu
from jax.experimental.pallas import tpu_sc as plsc

N = 10000
E = 320000
NODE = 128
EDGE = 16
HID = 32
NN = 100
NF = 10

NC = 2
NS = 16
NW = NC * NS
L = 16

EH = E // 2
SUB = 125
RW = EH // (NW * SUB)
NBUF = 4
NP = 10240
RPT = NP // NS
AW = HID + 8

_f32 = jnp.float32


def _node_proj_body(x_ref, wa_ref, wb_ref, b1_ref, xa_ref, xb_ref):
    xv = x_ref[...]
    xa_ref[0:N, :] = jnp.dot(xv, wa_ref[...], preferred_element_type=_f32, precision=jax.lax.Precision.HIGHEST) + b1_ref[...]
    xb_ref[0:N, :] = jnp.dot(xv, wb_ref[...], preferred_element_type=_f32, precision=jax.lax.Precision.HIGHEST)
    xa_ref[N:NP, :] = jnp.zeros((NP - N, HID), _f32)
    xb_ref[N:NP, :] = jnp.zeros((NP - N, HID), _f32)


def _node_proj(x, w1a, w1b, b1row):
    return pl.pallas_call(
        _node_proj_body,
        out_shape=(
            jax.ShapeDtypeStruct((NP, HID), _f32),
            jax.ShapeDtypeStruct((NP, HID), _f32),
        ),
    )(x, w1a, w1b, b1row)


_ER = EH // 8
_EB = 4000


def _edge_proj_body(attr_ref, wbd_ref, o_ref):
    o_ref[...] = jnp.dot(attr_ref[...], wbd_ref[...], preferred_element_type=_f32)


def _edge_proj(attr4, wbd):
    return pl.pallas_call(
        _edge_proj_body,
        grid=(_ER // _EB,),
        in_specs=[
            pl.BlockSpec((_EB, 8 * EDGE), lambda i: (i, 0)),
            pl.BlockSpec((8 * EDGE, 8 * HID), lambda i: (0, 0)),
        ],
        out_specs=pl.BlockSpec((_EB, 8 * HID), lambda i: (i, 0)),
        out_shape=jax.ShapeDtypeStruct((_ER, 8 * HID), _f32),
    )(attr4, wbd)


_sc_mesh = plsc.VectorSubcoreMesh(
    core_axis_name="c", subcore_axis_name="s", num_cores=NC, num_subcores=NS
)


@functools.partial(
    pl.kernel,
    out_type=jax.ShapeDtypeStruct((NC, NP, AW), _f32),
    mesh=_sc_mesh,
    compiler_params=pltpu.CompilerParams(use_tc_tiling_on_sc=False),
    scratch_types=[
        pltpu.VMEM((RW, SUB), jnp.int32),
        pltpu.VMEM((RW, SUB), jnp.int32),
        pltpu.VMEM((NBUF, SUB, HID), _f32),
        pltpu.VMEM((NBUF, SUB, HID), _f32),
        pltpu.VMEM((NBUF, SUB, HID), _f32),
        pltpu.VMEM((2, SUB, AW), _f32),
        pltpu.VMEM((RPT, AW), _f32),
        pltpu.VMEM_SHARED((NP, AW), _f32),
        pltpu.SemaphoreType.DMA,
        pltpu.SemaphoreType.DMA,
        pltpu.SemaphoreType.DMA,
        pltpu.SemaphoreType.DMA,
        pltpu.SemaphoreType.DMA,
        pltpu.SemaphoreType.DMA,
    ],
)
def _sc_edge(src_hbm, dst_hbm, xa_hbm, xb_hbm, ea_hbm, out_hbm,
             si, di, av, bv, ev, hv, st, acc,
             sem0, sem1, sem2, sem3, sems0, sems1):
    c = lax.axis_index("c")
    s = lax.axis_index("s")
    wid = c * NS + s
    sems = [sem0, sem1, sem2, sem3]
    ssems = [sems0, sems1]

    zvec = jnp.zeros((L,), _f32)

    def _zrow(i, _):
        for off in (0, L, AW - L):
            st[i, pl.ds(off, L)] = zvec
        return ()
    lax.fori_loop(0, RPT, _zrow, (), unroll=8)
    rsl = pl.ds(s * RPT, RPT)
    pltpu.sync_copy(st, acc.at[rsl])

    cvec = jnp.where(lax.iota(jnp.int32, L) == HID - (AW - L), 1.0,
                     0.0).astype(_f32)

    def _hrow(i, _):
        hv[0, i, pl.ds(AW - L, L)] = cvec
        hv[1, i, pl.ds(AW - L, L)] = cvec
        return ()
    lax.fori_loop(0, SUB, _hrow, (), unroll=8)

    pltpu.sync_copy(src_hbm.at[wid], si)
    pltpu.sync_copy(dst_hbm.at[wid], di)

    plsc.subcore_barrier()

    ebase = wid * RW * SUB

    def _fire(r, k):
        pltpu.async_copy(xa_hbm.at[si.at[r]], av.at[k], sems[k])
        pltpu.async_copy(xb_hbm.at[di.at[r]], bv.at[k], sems[k])
        pltpu.async_copy(ea_hbm.at[pl.ds(ebase + r * SUB, SUB)], ev.at[k],
                         sems[k])

    def _drain(r, k):
        pltpu.make_async_copy(xa_hbm.at[si.at[r]], av.at[k], sems[k]).wait()
        pltpu.make_async_copy(xb_hbm.at[di.at[r]], bv.at[k], sems[k]).wait()
        pltpu.make_async_copy(ea_hbm.at[pl.ds(ebase + r * SUB, SUB)],
                              ev.at[k], sems[k]).wait()

    def _compute_scatter(r, k):
        slot = k % 2

        @pl.when(r >= 2)
        def _():
            pltpu.make_async_copy(hv.at[slot], acc.at[si.at[r]],
                                  ssems[slot]).wait()

        def _row(i, _):
            for q in range(HID // L):
                sl = pl.ds(q * L, L)
                hv[slot, i, sl] = jnp.maximum(
                    av[k, i, sl] + bv[k, i, sl] + ev[k, i, sl], 0.0)
            return ()
        lax.fori_loop(0, SUB, _row, (), unroll=10)
        pltpu.async_copy(hv.at[slot], acc.at[si.at[r]], ssems[slot], add=True)

    for k in range(NBUF - 1):
        _fire(k, k)

    def _outer(t, _):
        for k in range(NBUF):
            r = t * NBUF + k
            kp = (k + NBUF - 1) % NBUF

            @pl.when(r + NBUF - 1 < RW)
            def _():
                _fire(r + NBUF - 1, kp)
            _drain(r, k)
            _compute_scatter(r, k)
        return ()
    lax.fori_loop(0, RW // NBUF, _outer, ())

    pltpu.make_async_copy(hv.at[0], acc.at[si.at[0]], ssems[0]).wait()
    pltpu.make_async_copy(hv.at[1], acc.at[si.at[0]], ssems[1]).wait()

    plsc.subcore_barrier()

    pltpu.sync_copy(acc.at[rsl], st)
    pltpu.sync_copy(st, out_hbm.at[c, rsl])


def _softplus(z):
    return jnp.maximum(z, 0.0) + jnp.log(1.0 + jnp.exp(-jnp.abs(z)))


def _tail_body(x_ref, p_ref, q_ref, w2_ref, b2_ref, whx_ref, wha_ref, bh_ref,
               high_ref, inv_ref, ord_ref):
    S = (p_ref[0, 0:N, 0:HID] + p_ref[1, 0:N, 0:HID]
         + q_ref[0, 0:N, 0:HID] + q_ref[1, 0:N, 0:HID])
    cnt = (p_ref[0, 0:N, HID:HID + 1] + p_ref[1, 0:N, HID:HID + 1]
           + q_ref[0, 0:N, HID:HID + 1] + q_ref[1, 0:N, HID:HID + 1])
    agg = jnp.dot(S, w2_ref[...], preferred_element_type=_f32, precision=jax.lax.Precision.HIGHEST) + cnt * b2_ref[...]
    heads = (jnp.dot(x_ref[...], whx_ref[...], preferred_element_type=_f32, precision=jax.lax.Precision.HIGHEST)
             + jnp.dot(agg, wha_ref[...], preferred_element_type=_f32, precision=jax.lax.Precision.HIGHEST)
             + bh_ref[...])
    conc = _softplus(heads[:, 0:1] + 1e-10)
    alpha = _softplus(heads[:, 1:2] + 1e-20) + 1e-20
    beta = _softplus(heads[:, 2:3] + 1e-20) + 1.0
    inv_ref[...] = conc / (jnp.sum(conc) + 1e-20)
    ord_ref[...] = alpha / (alpha + beta) * high_ref[0, 0]


def _tail(x, p1, p2, w2, b2row, whx, wha, bh, high11):
    return pl.pallas_call(
        _tail_body,
        out_shape=(
            jax.ShapeDtypeStruct((N, 1), _f32),
            jax.ShapeDtypeStruct((N, 1), _f32),
        ),
    )(x, p1, p2, w2, b2row, whx, wha, bh, high11)


def kernel(x, edge_index, edge_attr, W1, b1, W2, b2, Wc, bc, Wm, bm, Ws, bs,
           high, deterministic):
    w1a = W1[:NODE]
    w1b = W1[NODE:2 * NODE]
    w1c = W1[2 * NODE:]
    b1row = b1.reshape(1, HID)
    b2row = b2.reshape(1, HID)
    wh = jnp.concatenate([Wc, Wm, Ws], axis=1)
    wh = jnp.pad(wh, ((0, 0), (0, 5)))
    whx = wh[:NODE]
    wha = wh[NODE:]
    bh = jnp.pad(jnp.stack([bc[0], bm[0], bs[0]]), (0, 5)).reshape(1, 8)
    high11 = jnp.asarray(high, _f32).reshape(1, 1)

    wbd = jnp.kron(jnp.eye(8, dtype=_f32), w1c)
    srcs, dsts, attr4s = [], [], []
    for h in range(2):
        esl = slice(h * EH, (h + 1) * EH)
        srcs.append(edge_index[0, esl].reshape(NW, RW, SUB))
        dsts.append(edge_index[1, esl].reshape(NW, RW, SUB))
        attr4s.append(edge_attr[esl].reshape(_ER, 8 * EDGE))

    xa, xb = _node_proj(x, w1a, w1b, b1row)
    parts = []
    for h in range(2):
        ea_h = _edge_proj(attr4s[h], wbd).reshape(EH, HID)
        parts.append(_sc_edge(srcs[h], dsts[h], xa, xb, ea_h))
    inv, orda = _tail(x, parts[0], parts[1], W2, b2row, whx, wha, bh, high11)

    inventory_act = inv.reshape(NN, NN)
    order_act = orda.reshape(NN, NN)[:, NN - NF:].reshape(-1)
    return (inventory_act, order_act)


# speedup vs baseline: 1.2143x; 1.2143x over previous
import functools

import jax
import jax.numpy as jnp
from jax import lax
from jax.experimental import pallas as pl
from jax.experimental.pallas import tpu as pltpu
from jax.experimental.pallas import tpu_sc as plsc

N = 10000
E = 320000
NODE = 128
EDGE = 16
HID = 32
NN = 100
NF = 10

NC = 2
NS = 16
NW = NC * NS
L = 16

SUB = 125
RW = E // (NW * SUB)
NBUF = 4
NP = 10240
RPT = NP // NS
AW = HID + 8

_f32 = jnp.float32


def _node_proj_body(x_ref, wa_ref, wb_ref, b1_ref, xa_ref, xb_ref):
    xv = x_ref[...]
    xa_ref[0:N, :] = jnp.dot(xv, wa_ref[...], preferred_element_type=_f32, precision=jax.lax.Precision.HIGHEST) + b1_ref[...]
    xb_ref[0:N, :] = jnp.dot(xv, wb_ref[...], preferred_element_type=_f32, precision=jax.lax.Precision.HIGHEST)
    xa_ref[N:NP, :] = jnp.zeros((NP - N, HID), _f32)
    xb_ref[N:NP, :] = jnp.zeros((NP - N, HID), _f32)


def _node_proj(x, w1a, w1b, b1row):
    return pl.pallas_call(
        _node_proj_body,
        out_shape=(
            jax.ShapeDtypeStruct((NP, HID), _f32),
            jax.ShapeDtypeStruct((NP, HID), _f32),
        ),
    )(x, w1a, w1b, b1row)


_ER = E // 8
_EB = 4000


def _edge_proj_body(attr_ref, wbd_ref, o_ref):
    o_ref[...] = jnp.dot(attr_ref[...], wbd_ref[...], preferred_element_type=_f32)


def _edge_proj(attr4, wbd):
    return pl.pallas_call(
        _edge_proj_body,
        grid=(_ER // _EB,),
        in_specs=[
            pl.BlockSpec((_EB, 8 * EDGE), lambda i: (i, 0)),
            pl.BlockSpec((8 * EDGE, 8 * HID), lambda i: (0, 0)),
        ],
        out_specs=pl.BlockSpec((_EB, 8 * HID), lambda i: (i, 0)),
        out_shape=jax.ShapeDtypeStruct((_ER, 8 * HID), _f32),
    )(attr4, wbd)


_sc_mesh = plsc.VectorSubcoreMesh(
    core_axis_name="c", subcore_axis_name="s", num_cores=NC, num_subcores=NS
)


@functools.partial(
    pl.kernel,
    out_type=jax.ShapeDtypeStruct((NC, NP, AW), _f32),
    mesh=_sc_mesh,
    compiler_params=pltpu.CompilerParams(use_tc_tiling_on_sc=False),
    scratch_types=[
        pltpu.VMEM((RW, SUB), jnp.int32),
        pltpu.VMEM((RW, SUB), jnp.int32),
        pltpu.VMEM((NBUF, SUB, HID), _f32),
        pltpu.VMEM((NBUF, SUB, HID), _f32),
        pltpu.VMEM((NBUF, SUB, HID), _f32),
        pltpu.VMEM((2, SUB, AW), _f32),
        pltpu.VMEM((RPT, AW), _f32),
        pltpu.VMEM_SHARED((NP, AW), _f32),
        pltpu.SemaphoreType.DMA,
        pltpu.SemaphoreType.DMA,
        pltpu.SemaphoreType.DMA,
        pltpu.SemaphoreType.DMA,
        pltpu.SemaphoreType.DMA,
        pltpu.SemaphoreType.DMA,
    ],
)
def _sc_edge(src_hbm, dst_hbm, xa_hbm, xb_hbm, ea_hbm, out_hbm,
             si, di, av, bv, ev, hv, st, acc,
             sem0, sem1, sem2, sem3, sems0, sems1):
    c = lax.axis_index("c")
    s = lax.axis_index("s")
    wid = c * NS + s
    sems = [sem0, sem1, sem2, sem3]
    ssems = [sems0, sems1]

    zvec = jnp.zeros((L,), _f32)

    def _zrow(i, _):
        for off in (0, L, AW - L):
            st[i, pl.ds(off, L)] = zvec
        return ()
    lax.fori_loop(0, RPT, _zrow, (), unroll=8)
    rsl = pl.ds(s * RPT, RPT)
    pltpu.sync_copy(st, acc.at[rsl])

    cvec = jnp.where(lax.iota(jnp.int32, L) == HID - (AW - L), 1.0,
                     0.0).astype(_f32)

    def _hrow(i, _):
        hv[0, i, pl.ds(AW - L, L)] = cvec
        hv[1, i, pl.ds(AW - L, L)] = cvec
        return ()
    lax.fori_loop(0, SUB, _hrow, (), unroll=8)

    pltpu.sync_copy(src_hbm.at[wid], si)
    pltpu.sync_copy(dst_hbm.at[wid], di)

    plsc.subcore_barrier()

    ebase = wid * RW * SUB

    def _fire(r, k):
        pltpu.async_copy(xa_hbm.at[si.at[r]], av.at[k], sems[k])
        pltpu.async_copy(xb_hbm.at[di.at[r]], bv.at[k], sems[k])
        pltpu.async_copy(ea_hbm.at[pl.ds(ebase + r * SUB, SUB)], ev.at[k],
                         sems[k])

    def _drain(r, k):
        pltpu.make_async_copy(xa_hbm.at[si.at[r]], av.at[k], sems[k]).wait()
        pltpu.make_async_copy(xb_hbm.at[di.at[r]], bv.at[k], sems[k]).wait()
        pltpu.make_async_copy(ea_hbm.at[pl.ds(ebase + r * SUB, SUB)],
                              ev.at[k], sems[k]).wait()

    def _compute_scatter(r, k):
        slot = k % 2

        @pl.when(r >= 2)
        def _():
            pltpu.make_async_copy(hv.at[slot], acc.at[si.at[r]],
                                  ssems[slot]).wait()

        def _row(i, _):
            for q in range(HID // L):
                sl = pl.ds(q * L, L)
                hv[slot, i, sl] = jnp.maximum(
                    av[k, i, sl] + bv[k, i, sl] + ev[k, i, sl], 0.0)
            return ()
        lax.fori_loop(0, SUB, _row, (), unroll=10)
        pltpu.async_copy(hv.at[slot], acc.at[si.at[r]], ssems[slot], add=True)

    for k in range(NBUF - 1):
        _fire(k, k)

    def _outer(t, _):
        for k in range(NBUF):
            r = t * NBUF + k
            kp = (k + NBUF - 1) % NBUF

            @pl.when(r + NBUF - 1 < RW)
            def _():
                _fire(r + NBUF - 1, kp)
            _drain(r, k)
            _compute_scatter(r, k)
        return ()
    lax.fori_loop(0, RW // NBUF, _outer, ())

    pltpu.make_async_copy(hv.at[0], acc.at[si.at[0]], ssems[0]).wait()
    pltpu.make_async_copy(hv.at[1], acc.at[si.at[0]], ssems[1]).wait()

    plsc.subcore_barrier()

    pltpu.sync_copy(acc.at[rsl], st)
    pltpu.sync_copy(st, out_hbm.at[c, rsl])


def _softplus(z):
    return jnp.maximum(z, 0.0) + jnp.log(1.0 + jnp.exp(-jnp.abs(z)))


def _tail_body(x_ref, p_ref, w2_ref, b2_ref, whx_ref, wha_ref, bh_ref,
               high_ref, inv_ref, ord_ref):
    S = p_ref[0, 0:N, 0:HID] + p_ref[1, 0:N, 0:HID]
    cnt = p_ref[0, 0:N, HID:HID + 1] + p_ref[1, 0:N, HID:HID + 1]
    agg = jnp.dot(S, w2_ref[...], preferred_element_type=_f32, precision=jax.lax.Precision.HIGHEST) + cnt * b2_ref[...]
    heads = (jnp.dot(x_ref[...], whx_ref[...], preferred_element_type=_f32, precision=jax.lax.Precision.HIGHEST)
             + jnp.dot(agg, wha_ref[...], preferred_element_type=_f32, precision=jax.lax.Precision.HIGHEST)
             + bh_ref[...])
    conc = _softplus(heads[:, 0:1] + 1e-10)
    alpha = _softplus(heads[:, 1:2] + 1e-20) + 1e-20
    beta = _softplus(heads[:, 2:3] + 1e-20) + 1.0
    inv_ref[...] = conc / (jnp.sum(conc) + 1e-20)
    ord_ref[...] = alpha / (alpha + beta) * high_ref[0, 0]


def _tail(x, partials, w2, b2row, whx, wha, bh, high11):
    return pl.pallas_call(
        _tail_body,
        out_shape=(
            jax.ShapeDtypeStruct((N, 1), _f32),
            jax.ShapeDtypeStruct((N, 1), _f32),
        ),
    )(x, partials, w2, b2row, whx, wha, bh, high11)


def kernel(x, edge_index, edge_attr, W1, b1, W2, b2, Wc, bc, Wm, bm, Ws, bs,
           high, deterministic):
    w1a = W1[:NODE]
    w1b = W1[NODE:2 * NODE]
    w1c = W1[2 * NODE:]
    b1row = b1.reshape(1, HID)
    b2row = b2.reshape(1, HID)
    wh = jnp.concatenate([Wc, Wm, Ws], axis=1)
    wh = jnp.pad(wh, ((0, 0), (0, 5)))
    whx = wh[:NODE]
    wha = wh[NODE:]
    bh = jnp.pad(jnp.stack([bc[0], bm[0], bs[0]]), (0, 5)).reshape(1, 8)
    high11 = jnp.asarray(high, _f32).reshape(1, 1)

    wbd = jnp.kron(jnp.eye(8, dtype=_f32), w1c)
    src3 = edge_index[0].reshape(NW, RW, SUB)
    dst3 = edge_index[1].reshape(NW, RW, SUB)
    attr4 = edge_attr.reshape(_ER, 8 * EDGE)

    xa, xb = _node_proj(x, w1a, w1b, b1row)
    ea = _edge_proj(attr4, wbd).reshape(E, HID)
    partials = _sc_edge(src3, dst3, xa, xb, ea)
    inv, orda = _tail(x, partials, W2, b2row, whx, wha, bh, high11)

    inventory_act = inv.reshape(NN, NN)
    order_act = orda.reshape(NN, NN)[:, NN - NF:].reshape(-1)
    return (inventory_act, order_act)


# speedup vs baseline: 1.2442x; 1.0246x over previous
import functools

import jax
import jax.numpy as jnp
from jax import lax
from jax.experimental import pallas as pl
from jax.experimental.pallas import tpu as pltpu
from jax.experimental.pallas import tpu_sc as plsc

N = 10000
E = 320000
NODE = 128
EDGE = 16
HID = 32
NN = 100
NF = 10

NC = 2
NS = 16
NW = NC * NS
L = 16

SUB = 125
RW = E // (NW * SUB)
NBUF = 4
NP = 10240
RPT = NP // NS
AW = HID + 8

_f32 = jnp.float32


_N4 = N // 4
_NP4 = NP // 4


def _node_proj_body(x4_ref, wa_ref, wb_ref, b1_ref, xa_ref, xb_ref):
    xv = x4_ref[...]
    xa_ref[0:_N4, :] = jnp.dot(xv, wa_ref[...], preferred_element_type=_f32,
                               precision=jax.lax.Precision.HIGHEST) + b1_ref[...]
    xb_ref[0:_N4, :] = jnp.dot(xv, wb_ref[...], preferred_element_type=_f32,
                               precision=jax.lax.Precision.HIGHEST)
    xa_ref[_N4:_NP4, :] = jnp.zeros((_NP4 - _N4, 4 * HID), _f32)
    xb_ref[_N4:_NP4, :] = jnp.zeros((_NP4 - _N4, 4 * HID), _f32)


def _node_proj(x4, wa4, wb4, b1row4):
    return pl.pallas_call(
        _node_proj_body,
        out_shape=(
            jax.ShapeDtypeStruct((_NP4, 4 * HID), _f32),
            jax.ShapeDtypeStruct((_NP4, 4 * HID), _f32),
        ),
    )(x4, wa4, wb4, b1row4)


_ER = E // 8
_EB = 4000


def _edge_proj_body(attr_ref, wbd_ref, o_ref):
    o_ref[...] = jnp.dot(attr_ref[...], wbd_ref[...], preferred_element_type=_f32)


def _edge_proj(attr4, wbd):
    return pl.pallas_call(
        _edge_proj_body,
        grid=(_ER // _EB,),
        in_specs=[
            pl.BlockSpec((_EB, 8 * EDGE), lambda i: (i, 0)),
            pl.BlockSpec((8 * EDGE, 8 * HID), lambda i: (0, 0)),
        ],
        out_specs=pl.BlockSpec((_EB, 8 * HID), lambda i: (i, 0)),
        out_shape=jax.ShapeDtypeStruct((_ER, 8 * HID), _f32),
    )(attr4, wbd)


_sc_mesh = plsc.VectorSubcoreMesh(
    core_axis_name="c", subcore_axis_name="s", num_cores=NC, num_subcores=NS
)


@functools.partial(
    pl.kernel,
    out_type=jax.ShapeDtypeStruct((NC, NP, AW), _f32),
    mesh=_sc_mesh,
    compiler_params=pltpu.CompilerParams(use_tc_tiling_on_sc=False),
    scratch_types=[
        pltpu.VMEM((RW, SUB), jnp.int32),
        pltpu.VMEM((RW, SUB), jnp.int32),
        pltpu.VMEM((NBUF, SUB, HID), _f32),
        pltpu.VMEM((NBUF, SUB, HID), _f32),
        pltpu.VMEM((NBUF, SUB, HID), _f32),
        pltpu.VMEM((2, SUB, AW), _f32),
        pltpu.VMEM((RPT, AW), _f32),
        pltpu.VMEM_SHARED((NP, AW), _f32),
        pltpu.SemaphoreType.DMA,
        pltpu.SemaphoreType.DMA,
        pltpu.SemaphoreType.DMA,
        pltpu.SemaphoreType.DMA,
        pltpu.SemaphoreType.DMA,
        pltpu.SemaphoreType.DMA,
    ],
)
def _sc_edge(src_hbm, dst_hbm, xa_hbm, xb_hbm, ea_hbm, out_hbm,
             si, di, av, bv, ev, hv, st, acc,
             sem0, sem1, sem2, sem3, sems0, sems1):
    c = lax.axis_index("c")
    s = lax.axis_index("s")
    wid = c * NS + s
    sems = [sem0, sem1, sem2, sem3]
    ssems = [sems0, sems1]

    zvec = jnp.zeros((L,), _f32)

    def _zrow(i, _):
        for off in (0, L, AW - L):
            st[i, pl.ds(off, L)] = zvec
        return ()
    lax.fori_loop(0, RPT, _zrow, (), unroll=8)
    rsl = pl.ds(s * RPT, RPT)
    pltpu.sync_copy(st, acc.at[rsl])

    cvec = jnp.where(lax.iota(jnp.int32, L) == HID - (AW - L), 1.0,
                     0.0).astype(_f32)

    def _hrow(i, _):
        hv[0, i, pl.ds(AW - L, L)] = cvec
        hv[1, i, pl.ds(AW - L, L)] = cvec
        return ()
    lax.fori_loop(0, SUB, _hrow, (), unroll=8)

    pltpu.sync_copy(src_hbm.at[wid], si)
    pltpu.sync_copy(dst_hbm.at[wid], di)

    plsc.subcore_barrier()

    ebase = wid * RW * SUB

    def _fire(r, k):
        pltpu.async_copy(xa_hbm.at[si.at[r]], av.at[k], sems[k])
        pltpu.async_copy(xb_hbm.at[di.at[r]], bv.at[k], sems[k])
        pltpu.async_copy(ea_hbm.at[pl.ds(ebase + r * SUB, SUB)], ev.at[k],
                         sems[k])

    def _drain(r, k):
        pltpu.make_async_copy(xa_hbm.at[si.at[r]], av.at[k], sems[k]).wait()
        pltpu.make_async_copy(xb_hbm.at[di.at[r]], bv.at[k], sems[k]).wait()
        pltpu.make_async_copy(ea_hbm.at[pl.ds(ebase + r * SUB, SUB)],
                              ev.at[k], sems[k]).wait()

    def _compute_scatter(r, k):
        slot = k % 2

        @pl.when(r >= 2)
        def _():
            pltpu.make_async_copy(hv.at[slot], acc.at[si.at[r]],
                                  ssems[slot]).wait()

        def _row(i, _):
            for q in range(HID // L):
                sl = pl.ds(q * L, L)
                hv[slot, i, sl] = jnp.maximum(
                    av[k, i, sl] + bv[k, i, sl] + ev[k, i, sl], 0.0)
            return ()
        lax.fori_loop(0, SUB, _row, (), unroll=10)
        pltpu.async_copy(hv.at[slot], acc.at[si.at[r]], ssems[slot], add=True)

    for k in range(NBUF - 1):
        _fire(k, k)

    def _outer(t, _):
        for k in range(NBUF):
            r = t * NBUF + k
            kp = (k + NBUF - 1) % NBUF

            @pl.when(r + NBUF - 1 < RW)
            def _():
                _fire(r + NBUF - 1, kp)
            _drain(r, k)
            _compute_scatter(r, k)
        return ()
    lax.fori_loop(0, RW // NBUF, _outer, ())

    pltpu.make_async_copy(hv.at[0], acc.at[si.at[0]], ssems[0]).wait()
    pltpu.make_async_copy(hv.at[1], acc.at[si.at[0]], ssems[1]).wait()

    plsc.subcore_barrier()

    pltpu.sync_copy(acc.at[rsl], st)
    pltpu.sync_copy(st, out_hbm.at[c, rsl])


def _softplus(z):
    return jnp.maximum(z, 0.0) + jnp.log(1.0 + jnp.exp(-jnp.abs(z)))


def _tail_body(x_ref, p_ref, w2_ref, b2_ref, whx_ref, wha_ref, bh_ref,
               high_ref, inv_ref, ord_ref):
    S = p_ref[0, 0:N, 0:HID] + p_ref[1, 0:N, 0:HID]
    cnt = p_ref[0, 0:N, HID:HID + 1] + p_ref[1, 0:N, HID:HID + 1]
    agg = jnp.dot(S, w2_ref[...], preferred_element_type=_f32, precision=jax.lax.Precision.HIGHEST) + cnt * b2_ref[...]
    heads = (jnp.dot(x_ref[...], whx_ref[...], preferred_element_type=_f32, precision=jax.lax.Precision.HIGHEST)
             + jnp.dot(agg, wha_ref[...], preferred_element_type=_f32, precision=jax.lax.Precision.HIGHEST)
             + bh_ref[...])
    conc = _softplus(heads[:, 0:1] + 1e-10)
    alpha = _softplus(heads[:, 1:2] + 1e-20) + 1e-20
    beta = _softplus(heads[:, 2:3] + 1e-20) + 1.0
    inv_ref[...] = conc / (jnp.sum(conc) + 1e-20)
    ord_ref[...] = alpha / (alpha + beta) * high_ref[0, 0]


def _tail(x, partials, w2, b2row, whx, wha, bh, high11):
    return pl.pallas_call(
        _tail_body,
        out_shape=(
            jax.ShapeDtypeStruct((N, 1), _f32),
            jax.ShapeDtypeStruct((N, 1), _f32),
        ),
    )(x, partials, w2, b2row, whx, wha, bh, high11)


def kernel(x, edge_index, edge_attr, W1, b1, W2, b2, Wc, bc, Wm, bm, Ws, bs,
           high, deterministic):
    w1a = W1[:NODE]
    w1b = W1[NODE:2 * NODE]
    w1c = W1[2 * NODE:]
    b1row = b1.reshape(1, HID)
    b2row = b2.reshape(1, HID)
    wh = jnp.concatenate([Wc, Wm, Ws], axis=1)
    wh = jnp.pad(wh, ((0, 0), (0, 5)))
    whx = wh[:NODE]
    wha = wh[NODE:]
    bh = jnp.pad(jnp.stack([bc[0], bm[0], bs[0]]), (0, 5)).reshape(1, 8)
    high11 = jnp.asarray(high, _f32).reshape(1, 1)

    wbd = jnp.kron(jnp.eye(8, dtype=_f32), w1c)
    wa4 = jnp.kron(jnp.eye(4, dtype=_f32), w1a)
    wb4 = jnp.kron(jnp.eye(4, dtype=_f32), w1b)
    b1row4 = jnp.tile(b1, 4).reshape(1, 4 * HID)
    src3 = edge_index[0].reshape(NW, RW, SUB)
    dst3 = edge_index[1].reshape(NW, RW, SUB)
    attr4 = edge_attr.reshape(_ER, 8 * EDGE)
    x4 = x.reshape(_N4, 4 * NODE)

    xa4, xb4 = _node_proj(x4, wa4, wb4, b1row4)
    xa = xa4.reshape(NP, HID)
    xb = xb4.reshape(NP, HID)
    ea = _edge_proj(attr4, wbd).reshape(E, HID)
    partials = _sc_edge(src3, dst3, xa, xb, ea)
    inv, orda = _tail(x, partials, W2, b2row, whx, wha, bh, high11)

    inventory_act = inv.reshape(NN, NN)
    order_act = orda.reshape(NN, NN)[:, NN - NF:].reshape(-1)
    return (inventory_act, order_act)
